# baseline (speedup 1.0000x reference)
import jax
import jax.numpy as jnp
from jax import lax
from jax.experimental import pallas as pl
from jax.experimental.pallas import tpu as pltpu
from jax.experimental.pallas import tpu_sc as plsc

N = 10000
E = 320000
H = 128
ED = 16

_NC = 2
_NS = 16
_NW = _NC * _NS

_SLICES = ((0, 128000), (128000, 128000), (256000, 64000))
_CHUNK = 200
_CHUNK_SC = 80

_RPT = 624


def _make_gather_body(off, span):
    nchunk = span // _CHUNK

    def _gather_body(h_hbm, src_hbm, dst_hbm, zeros_n_hbm, ones_hbm,
                     out_s, out_d, hist_out,
                     idx_s0, idx_s1, idx_d0, idx_d1,
                     rows_s0, rows_s1, rows_d0, rows_d1,
                     ones_v, sem_gs0, sem_gs1, sem_gd0, sem_gd1,
                     sem_ws0, sem_ws1, sem_wd0, sem_wd1, hist_sh):
        cid = lax.axis_index("c")
        sid = lax.axis_index("s")
        wid = sid * _NC + cid
        base = wid * span

        idx_s = (idx_s0, idx_s1)
        idx_d = (idx_d0, idx_d1)
        rows_s = (rows_s0, rows_s1)
        rows_d = (rows_d0, rows_d1)
        sem_gs = (sem_gs0, sem_gs1)
        sem_gd = (sem_gd0, sem_gd1)
        sem_ws = (sem_ws0, sem_ws1)
        sem_wd = (sem_wd0, sem_wd1)

        @pl.when(sid == 0)
        def _():
            pltpu.sync_copy(zeros_n_hbm, hist_sh)

        pltpu.sync_copy(ones_hbm, ones_v)
        plsc.subcore_barrier()

        ws_pend = {}
        wd_pend = {}
        for c in range(nchunk):
            sl = c % 2
            loff = base + c * _CHUNK
            goff = off + loff
            if c >= 2:
                ws_pend.pop(sl).wait()
                wd_pend.pop(sl).wait()
            pltpu.sync_copy(src_hbm.at[pl.ds(goff, _CHUNK)], idx_s[sl])
            pltpu.sync_copy(dst_hbm.at[pl.ds(goff, _CHUNK)], idx_d[sl])
            pltpu.sync_copy(ones_v, hist_sh.at[idx_d[sl]], add=True)
            gs = pltpu.async_copy(h_hbm.at[idx_s[sl]], rows_s[sl], sem_gs[sl])
            gd = pltpu.async_copy(h_hbm.at[idx_d[sl]], rows_d[sl], sem_gd[sl])
            gs.wait()
            ws_pend[sl] = pltpu.async_copy(
                rows_s[sl], out_s.at[pl.ds(loff, _CHUNK)], sem_ws[sl])
            gd.wait()
            wd_pend[sl] = pltpu.async_copy(
                rows_d[sl], out_d.at[pl.ds(loff, _CHUNK)], sem_wd[sl])
        for sl in list(ws_pend):
            ws_pend.pop(sl).wait()
            wd_pend.pop(sl).wait()

        plsc.subcore_barrier()

        @pl.when(sid == 0)
        def _():
            pltpu.sync_copy(hist_sh, hist_out.at[cid, 0])

    return _gather_body


def _sc_gather(off, ek, h2, src_idx, dst_idx, zeros_n, ones_c):
    run = pl.kernel(
        _make_gather_body(off, ek // _NW),
        out_type=(
            jax.ShapeDtypeStruct((ek, H), jnp.float32),
            jax.ShapeDtypeStruct((ek, H), jnp.float32),
            jax.ShapeDtypeStruct((_NC, 1, N), jnp.float32),
        ),
        mesh=plsc.VectorSubcoreMesh(core_axis_name="c", subcore_axis_name="s"),
        scratch_types=(
            [pltpu.VMEM((_CHUNK,), jnp.int32) for _ in range(4)]
            + [pltpu.VMEM((_CHUNK, H), jnp.float32) for _ in range(4)]
            + [pltpu.VMEM((_CHUNK,), jnp.float32)]
            + [pltpu.SemaphoreType.DMA for _ in range(8)]
            + [pltpu.VMEM_SHARED((N,), jnp.float32)]
        ),
    )
    return run(h2, src_idx, dst_idx, zeros_n, ones_c)


def _make_scatter_body(off, span):
    nchunk = span // _CHUNK_SC

    def _scatter_body(msg_hbm, dst_hbm, zeros_nh_hbm, agg_out,
                      idx0, idx1, rows0, rows1, sem_m0, sem_m1, agg_sh):
        idx_b = (idx0, idx1)
        rows = (rows0, rows1)
        sem_m = (sem_m0, sem_m1)
        cid = lax.axis_index("c")
        sid = lax.axis_index("s")
        wid = sid * _NC + cid
        base = wid * span

        r0 = pl.multiple_of(sid * _RPT, 8)

        @pl.when(sid < _NS - 1)
        def _():
            pltpu.sync_copy(zeros_nh_hbm.at[pl.ds(r0, _RPT)],
                            agg_sh.at[pl.ds(r0, _RPT)])

        @pl.when(sid == _NS - 1)
        def _():
            last = N - (_NS - 1) * _RPT
            pltpu.sync_copy(zeros_nh_hbm.at[pl.ds(r0, last)],
                            agg_sh.at[pl.ds(r0, last)])

        plsc.subcore_barrier()

        m_pend = {0: pltpu.async_copy(
            msg_hbm.at[pl.ds(base, _CHUNK_SC)], rows[0], sem_m[0])}
        pltpu.sync_copy(dst_hbm.at[pl.ds(off + base, _CHUNK_SC)], idx_b[0])
        for c in range(nchunk):
            sl = c % 2
            if c + 1 < nchunk:
                nx = (c + 1) % 2
                loff2 = base + (c + 1) * _CHUNK_SC
                m_pend[nx] = pltpu.async_copy(
                    msg_hbm.at[pl.ds(loff2, _CHUNK_SC)], rows[nx], sem_m[nx])
                pltpu.sync_copy(dst_hbm.at[pl.ds(off + loff2, _CHUNK_SC)],
                                idx_b[nx])
            m_pend.pop(sl).wait()
            pltpu.sync_copy(rows[sl], agg_sh.at[idx_b[sl]], add=True)
        plsc.subcore_barrier()

        @pl.when(sid < _NS - 1)
        def _():
            pltpu.sync_copy(agg_sh.at[pl.ds(r0, _RPT)],
                            agg_out.at[cid, pl.ds(r0, _RPT)])

        @pl.when(sid == _NS - 1)
        def _():
            last = N - (_NS - 1) * _RPT
            pltpu.sync_copy(agg_sh.at[pl.ds(r0, last)],
                            agg_out.at[cid, pl.ds(r0, last)])

    return _scatter_body


def _sc_scatter(off, ek, msg, dst_idx, zeros_nh):
    run = pl.kernel(
        _make_scatter_body(off, ek // _NW),
        out_type=jax.ShapeDtypeStruct((_NC, N, H), jnp.float32),
        mesh=plsc.VectorSubcoreMesh(core_axis_name="c", subcore_axis_name="s"),
        scratch_types=(
            [pltpu.VMEM((_CHUNK_SC,), jnp.int32) for _ in range(2)]
            + [pltpu.VMEM((_CHUNK_SC, H), jnp.float32) for _ in range(2)]
            + [pltpu.SemaphoreType.DMA for _ in range(2)]
            + [pltpu.VMEM_SHARED((N, H), jnp.float32)]
        ),
    )
    return run(msg, dst_idx, zeros_nh)


_SQRT_HALF = 0.7071067811865476


def _gelu(x):
    return 0.5 * x * (1.0 + lax.erf(x * _SQRT_HALF))


def _edge_mlp_body(hs_ref, hd_ref, ea_ref, w1s_ref, w1d_ref, w1e_ref, b1_ref,
                   w2_ref, b2_ref, wsrc_ref, bsrc_ref, out_ref):
    hs = hs_ref[...].astype(jnp.bfloat16)
    hd = hd_ref[...].astype(jnp.bfloat16)
    ea = ea_ref[...]
    z = (jnp.dot(hs, w1s_ref[...], preferred_element_type=jnp.float32)
         + jnp.dot(hd, w1d_ref[...], preferred_element_type=jnp.float32)
         + jnp.dot(ea, w1e_ref[...], preferred_element_type=jnp.float32)
         + b1_ref[...])
    hid = _gelu(z).astype(jnp.bfloat16)
    ctx = jnp.dot(hid, w2_ref[...], preferred_element_type=jnp.float32) + b2_ref[...]
    gate = jax.nn.sigmoid(ctx[:, :H])
    shift = ctx[:, H:]
    s = jnp.dot(hs, wsrc_ref[...], preferred_element_type=jnp.float32) + bsrc_ref[...]
    out_ref[...] = gate * s + shift


_BE = 3200


def _edge_mlp(off, ek, hs, hd, ea, w1s, w1d, w1e, b1, w2, b2, wsrc, bsrc):
    grid = ek // _BE
    kb = off // _BE
    full = lambda i: (0, 0)
    return pl.pallas_call(
        _edge_mlp_body,
        grid=(grid,),
        in_specs=[
            pl.BlockSpec((_BE, H), lambda i: (i, 0)),
            pl.BlockSpec((_BE, H), lambda i: (i, 0)),
            pl.BlockSpec((_BE, ED), lambda i: (i + kb, 0)),
            pl.BlockSpec((H, H), full),
            pl.BlockSpec((H, H), full),
            pl.BlockSpec((ED, H), full),
            pl.BlockSpec((1, H), full),
            pl.BlockSpec((H, 2 * H), full),
            pl.BlockSpec((1, 2 * H), full),
            pl.BlockSpec((H, H), full),
            pl.BlockSpec((1, H), full),
        ],
        out_specs=pl.BlockSpec((_BE, H), lambda i: (i, 0)),
        out_shape=jax.ShapeDtypeStruct((ek, H), jnp.float32),
    )(hs, hd, ea, w1s, w1d, w1e, b1, w2, b2, wsrc, bsrc)


def _node_body(h_ref, a00_ref, a01_ref, a10_ref, a11_ref, a20_ref, a21_ref,
               cnt_ref,
               wself_ref, bself_ref, wagg_ref, bagg_ref, gamma_ref, beta_ref,
               out_ref):
    h = h_ref[...]
    agg_sum = (a00_ref[0] + a01_ref[0] + a10_ref[0] + a11_ref[0]
               + a20_ref[0] + a21_ref[0])
    cnt = cnt_ref[...]
    indeg = jnp.maximum(cnt, 1.0)
    agg = agg_sum / indeg
    upd = (jnp.dot(h, wself_ref[...], preferred_element_type=jnp.float32)
           + bself_ref[...]
           + jnp.dot(agg, wagg_ref[...], preferred_element_type=jnp.float32)
           + bagg_ref[...])
    x = h + _gelu(upd)
    mu = jnp.mean(x, axis=1, keepdims=True)
    var = jnp.mean((x - mu) ** 2, axis=1, keepdims=True)
    out_ref[...] = (x - mu) * lax.rsqrt(var + 1e-5) * gamma_ref[...] + beta_ref[...]


_BN = 2000


def _node_update(h2, agg_a, agg_b, agg_c, cnt, wself, bself, wagg, bagg,
                 gamma, beta):
    grid = N // _BN
    full = lambda i: (0, 0)
    part0 = pl.BlockSpec((1, _BN, H), lambda i: (0, i, 0))
    part1 = pl.BlockSpec((1, _BN, H), lambda i: (1, i, 0))
    return pl.pallas_call(
        _node_body,
        grid=(grid,),
        in_specs=[
            pl.BlockSpec((_BN, H), lambda i: (i, 0)),
            part0, part1,
            part0, part1,
            part0, part1,
            pl.BlockSpec((_BN, 1), lambda i: (i, 0)),
            pl.BlockSpec((H, H), full),
            pl.BlockSpec((1, H), full),
            pl.BlockSpec((H, H), full),
            pl.BlockSpec((1, H), full),
            pl.BlockSpec((1, H), full),
            pl.BlockSpec((1, H), full),
        ],
        out_specs=pl.BlockSpec((_BN, H), lambda i: (i, 0)),
        out_shape=jax.ShapeDtypeStruct((N, H), jnp.float32),
    )(h2, agg_a, agg_a, agg_b, agg_b, agg_c, agg_c, cnt,
      wself, bself, wagg, bagg, gamma, beta)


def kernel(h, edge_attr, src_idx, dst_idx, Wsrc, bsrc, W1, b1, W2, b2,
           Wself, bself, Wagg, bagg, gamma, beta):
    h2 = h[0]
    ea = edge_attr[0]
    bf = jnp.bfloat16
    w1s = W1[:H].astype(bf)
    w1d = W1[H:2 * H].astype(bf)
    w1e = W1[2 * H:]

    zeros_nh = jnp.zeros((N, H), jnp.float32)
    zeros_n = jnp.zeros((N,), jnp.float32)
    ones_c = jnp.ones((_CHUNK,), jnp.float32)

    aggs = []
    hps = []
    for off, ek in _SLICES:
        hs, hd, hp = _sc_gather(off, ek, h2, src_idx, dst_idx, zeros_n, ones_c)
        hps.append(hp)
        msg = _edge_mlp(off, ek, hs, hd, ea, w1s, w1d, w1e, b1[None, :],
                        W2.astype(bf), b2[None, :], Wsrc.astype(bf),
                        bsrc[None, :])
        aggs.append(_sc_scatter(off, ek, msg, dst_idx, zeros_nh))

    cnt = (hps[0][0, 0] + hps[0][1, 0] + hps[1][0, 0] + hps[1][1, 0]
           + hps[2][0, 0] + hps[2][1, 0])[:, None]
    out = _node_update(h2, aggs[0], aggs[1], aggs[2], cnt,
                       Wself, bself[None, :], Wagg, bagg[None, :],
                       gamma[None, :], beta[None, :])
    return out[None]


# speedup vs baseline: 1.0084x; 1.0084x over previous
import jax
import jax.numpy as jnp
from jax import lax
from jax.experimental import pallas as pl
from jax.experimental.pallas import tpu as pltpu
from jax.experimental.pallas import tpu_sc as plsc

N = 10000
E = 320000
H = 128
ED = 16

_NC = 2
_NS = 16
_NW = _NC * _NS

_SLICES = ((0, 76800), (76800, 140800), (217600, 102400))
_CHUNK = 200
_CHUNK_SC = 80

_RPT = 624


def _make_gather_body(off, span):
    nchunk = span // _CHUNK

    def _gather_body(h_hbm, src_hbm, dst_hbm, zeros_n_hbm, ones_hbm,
                     out_s, out_d, hist_out,
                     idx_s0, idx_s1, idx_d0, idx_d1,
                     rows_s0, rows_s1, rows_d0, rows_d1,
                     ones_v, sem_gs0, sem_gs1, sem_gd0, sem_gd1,
                     sem_ws0, sem_ws1, sem_wd0, sem_wd1, hist_sh):
        cid = lax.axis_index("c")
        sid = lax.axis_index("s")
        wid = sid * _NC + cid
        base = wid * span

        idx_s = (idx_s0, idx_s1)
        idx_d = (idx_d0, idx_d1)
        rows_s = (rows_s0, rows_s1)
        rows_d = (rows_d0, rows_d1)
        sem_gs = (sem_gs0, sem_gs1)
        sem_gd = (sem_gd0, sem_gd1)
        sem_ws = (sem_ws0, sem_ws1)
        sem_wd = (sem_wd0, sem_wd1)

        @pl.when(sid == 0)
        def _():
            pltpu.sync_copy(zeros_n_hbm, hist_sh)

        pltpu.sync_copy(ones_hbm, ones_v)
        plsc.subcore_barrier()

        ws_pend = {}
        wd_pend = {}
        for c in range(nchunk):
            sl = c % 2
            loff = base + c * _CHUNK
            goff = off + loff
            if c >= 2:
                ws_pend.pop(sl).wait()
                wd_pend.pop(sl).wait()
            pltpu.sync_copy(src_hbm.at[pl.ds(goff, _CHUNK)], idx_s[sl])
            pltpu.sync_copy(dst_hbm.at[pl.ds(goff, _CHUNK)], idx_d[sl])
            pltpu.sync_copy(ones_v, hist_sh.at[idx_d[sl]], add=True)
            gs = pltpu.async_copy(h_hbm.at[idx_s[sl]], rows_s[sl], sem_gs[sl])
            gd = pltpu.async_copy(h_hbm.at[idx_d[sl]], rows_d[sl], sem_gd[sl])
            gs.wait()
            ws_pend[sl] = pltpu.async_copy(
                rows_s[sl], out_s.at[pl.ds(loff, _CHUNK)], sem_ws[sl])
            gd.wait()
            wd_pend[sl] = pltpu.async_copy(
                rows_d[sl], out_d.at[pl.ds(loff, _CHUNK)], sem_wd[sl])
        for sl in list(ws_pend):
            ws_pend.pop(sl).wait()
            wd_pend.pop(sl).wait()

        plsc.subcore_barrier()

        @pl.when(sid == 0)
        def _():
            pltpu.sync_copy(hist_sh, hist_out.at[cid, 0])

    return _gather_body


def _sc_gather(off, ek, h2, src_idx, dst_idx, zeros_n, ones_c):
    run = pl.kernel(
        _make_gather_body(off, ek // _NW),
        out_type=(
            jax.ShapeDtypeStruct((ek, H), jnp.float32),
            jax.ShapeDtypeStruct((ek, H), jnp.float32),
            jax.ShapeDtypeStruct((_NC, 1, N), jnp.float32),
        ),
        mesh=plsc.VectorSubcoreMesh(core_axis_name="c", subcore_axis_name="s"),
        scratch_types=(
            [pltpu.VMEM((_CHUNK,), jnp.int32) for _ in range(4)]
            + [pltpu.VMEM((_CHUNK, H), jnp.float32) for _ in range(4)]
            + [pltpu.VMEM((_CHUNK,), jnp.float32)]
            + [pltpu.SemaphoreType.DMA for _ in range(8)]
            + [pltpu.VMEM_SHARED((N,), jnp.float32)]
        ),
    )
    return run(h2, src_idx, dst_idx, zeros_n, ones_c)


def _make_scatter_body(off, span):
    nchunk = span // _CHUNK_SC

    def _scatter_body(msg_hbm, dst_hbm, zeros_nh_hbm, agg_out,
                      idx0, idx1, rows0, rows1, sem_m0, sem_m1, agg_sh):
        idx_b = (idx0, idx1)
        rows = (rows0, rows1)
        sem_m = (sem_m0, sem_m1)
        cid = lax.axis_index("c")
        sid = lax.axis_index("s")
        wid = sid * _NC + cid
        base = wid * span

        r0 = pl.multiple_of(sid * _RPT, 8)

        @pl.when(sid < _NS - 1)
        def _():
            pltpu.sync_copy(zeros_nh_hbm.at[pl.ds(r0, _RPT)],
                            agg_sh.at[pl.ds(r0, _RPT)])

        @pl.when(sid == _NS - 1)
        def _():
            last = N - (_NS - 1) * _RPT
            pltpu.sync_copy(zeros_nh_hbm.at[pl.ds(r0, last)],
                            agg_sh.at[pl.ds(r0, last)])

        plsc.subcore_barrier()

        m_pend = {0: pltpu.async_copy(
            msg_hbm.at[pl.ds(base, _CHUNK_SC)], rows[0], sem_m[0])}
        pltpu.sync_copy(dst_hbm.at[pl.ds(off + base, _CHUNK_SC)], idx_b[0])
        for c in range(nchunk):
            sl = c % 2
            if c + 1 < nchunk:
                nx = (c + 1) % 2
                loff2 = base + (c + 1) * _CHUNK_SC
                m_pend[nx] = pltpu.async_copy(
                    msg_hbm.at[pl.ds(loff2, _CHUNK_SC)], rows[nx], sem_m[nx])
                pltpu.sync_copy(dst_hbm.at[pl.ds(off + loff2, _CHUNK_SC)],
                                idx_b[nx])
            m_pend.pop(sl).wait()
            pltpu.sync_copy(rows[sl], agg_sh.at[idx_b[sl]], add=True)
        plsc.subcore_barrier()

        @pl.when(sid < _NS - 1)
        def _():
            pltpu.sync_copy(agg_sh.at[pl.ds(r0, _RPT)],
                            agg_out.at[cid, pl.ds(r0, _RPT)])

        @pl.when(sid == _NS - 1)
        def _():
            last = N - (_NS - 1) * _RPT
            pltpu.sync_copy(agg_sh.at[pl.ds(r0, last)],
                            agg_out.at[cid, pl.ds(r0, last)])

    return _scatter_body


def _sc_scatter(off, ek, msg, dst_idx, zeros_nh):
    run = pl.kernel(
        _make_scatter_body(off, ek // _NW),
        out_type=jax.ShapeDtypeStruct((_NC, N, H), jnp.float32),
        mesh=plsc.VectorSubcoreMesh(core_axis_name="c", subcore_axis_name="s"),
        scratch_types=(
            [pltpu.VMEM((_CHUNK_SC,), jnp.int32) for _ in range(2)]
            + [pltpu.VMEM((_CHUNK_SC, H), jnp.float32) for _ in range(2)]
            + [pltpu.SemaphoreType.DMA for _ in range(2)]
            + [pltpu.VMEM_SHARED((N, H), jnp.float32)]
        ),
    )
    return run(msg, dst_idx, zeros_nh)


_SQRT_HALF = 0.7071067811865476


def _gelu(x):
    return 0.5 * x * (1.0 + lax.erf(x * _SQRT_HALF))


def _edge_mlp_body(hs_ref, hd_ref, ea_ref, w1s_ref, w1d_ref, w1e_ref, b1_ref,
                   w2_ref, b2_ref, wsrc_ref, bsrc_ref, out_ref):
    hs = hs_ref[...].astype(jnp.bfloat16)
    hd = hd_ref[...].astype(jnp.bfloat16)
    ea = ea_ref[...]
    z = (jnp.dot(hs, w1s_ref[...], preferred_element_type=jnp.float32)
         + jnp.dot(hd, w1d_ref[...], preferred_element_type=jnp.float32)
         + jnp.dot(ea, w1e_ref[...], preferred_element_type=jnp.float32)
         + b1_ref[...])
    hid = _gelu(z).astype(jnp.bfloat16)
    ctx = jnp.dot(hid, w2_ref[...], preferred_element_type=jnp.float32) + b2_ref[...]
    gate = jax.nn.sigmoid(ctx[:, :H])
    shift = ctx[:, H:]
    s = jnp.dot(hs, wsrc_ref[...], preferred_element_type=jnp.float32) + bsrc_ref[...]
    out_ref[...] = gate * s + shift


_BE = 2560


def _edge_mlp(off, ek, hs, hd, ea, w1s, w1d, w1e, b1, w2, b2, wsrc, bsrc):
    grid = ek // _BE
    kb = off // _BE
    full = lambda i: (0, 0)
    return pl.pallas_call(
        _edge_mlp_body,
        grid=(grid,),
        in_specs=[
            pl.BlockSpec((_BE, H), lambda i: (i, 0)),
            pl.BlockSpec((_BE, H), lambda i: (i, 0)),
            pl.BlockSpec((_BE, ED), lambda i: (i + kb, 0)),
            pl.BlockSpec((H, H), full),
            pl.BlockSpec((H, H), full),
            pl.BlockSpec((ED, H), full),
            pl.BlockSpec((1, H), full),
            pl.BlockSpec((H, 2 * H), full),
            pl.BlockSpec((1, 2 * H), full),
            pl.BlockSpec((H, H), full),
            pl.BlockSpec((1, H), full),
        ],
        out_specs=pl.BlockSpec((_BE, H), lambda i: (i, 0)),
        out_shape=jax.ShapeDtypeStruct((ek, H), jnp.float32),
    )(hs, hd, ea, w1s, w1d, w1e, b1, w2, b2, wsrc, bsrc)


def _node_body(h_ref, a00_ref, a01_ref, a10_ref, a11_ref, a20_ref, a21_ref,
               cnt_ref,
               wself_ref, bself_ref, wagg_ref, bagg_ref, gamma_ref, beta_ref,
               out_ref):
    h = h_ref[...]
    agg_sum = (a00_ref[0] + a01_ref[0] + a10_ref[0] + a11_ref[0]
               + a20_ref[0] + a21_ref[0])
    cnt = cnt_ref[...]
    indeg = jnp.maximum(cnt, 1.0)
    agg = agg_sum / indeg
    upd = (jnp.dot(h, wself_ref[...], preferred_element_type=jnp.float32)
           + bself_ref[...]
           + jnp.dot(agg, wagg_ref[...], preferred_element_type=jnp.float32)
           + bagg_ref[...])
    x = h + _gelu(upd)
    mu = jnp.mean(x, axis=1, keepdims=True)
    var = jnp.mean((x - mu) ** 2, axis=1, keepdims=True)
    out_ref[...] = (x - mu) * lax.rsqrt(var + 1e-5) * gamma_ref[...] + beta_ref[...]


_BN = 2000


def _node_update(h2, agg_a, agg_b, agg_c, cnt, wself, bself, wagg, bagg,
                 gamma, beta):
    grid = N // _BN
    full = lambda i: (0, 0)
    part0 = pl.BlockSpec((1, _BN, H), lambda i: (0, i, 0))
    part1 = pl.BlockSpec((1, _BN, H), lambda i: (1, i, 0))
    return pl.pallas_call(
        _node_body,
        grid=(grid,),
        in_specs=[
            pl.BlockSpec((_BN, H), lambda i: (i, 0)),
            part0, part1,
            part0, part1,
            part0, part1,
            pl.BlockSpec((_BN, 1), lambda i: (i, 0)),
            pl.BlockSpec((H, H), full),
            pl.BlockSpec((1, H), full),
            pl.BlockSpec((H, H), full),
            pl.BlockSpec((1, H), full),
            pl.BlockSpec((1, H), full),
            pl.BlockSpec((1, H), full),
        ],
        out_specs=pl.BlockSpec((_BN, H), lambda i: (i, 0)),
        out_shape=jax.ShapeDtypeStruct((N, H), jnp.float32),
    )(h2, agg_a, agg_a, agg_b, agg_b, agg_c, agg_c, cnt,
      wself, bself, wagg, bagg, gamma, beta)


def kernel(h, edge_attr, src_idx, dst_idx, Wsrc, bsrc, W1, b1, W2, b2,
           Wself, bself, Wagg, bagg, gamma, beta):
    h2 = h[0]
    ea = edge_attr[0]
    bf = jnp.bfloat16
    w1s = W1[:H].astype(bf)
    w1d = W1[H:2 * H].astype(bf)
    w1e = W1[2 * H:]

    zeros_nh = jnp.zeros((N, H), jnp.float32)
    zeros_n = jnp.zeros((N,), jnp.float32)
    ones_c = jnp.ones((_CHUNK,), jnp.float32)

    aggs = []
    hps = []
    for off, ek in _SLICES:
        hs, hd, hp = _sc_gather(off, ek, h2, src_idx, dst_idx, zeros_n, ones_c)
        hps.append(hp)
        msg = _edge_mlp(off, ek, hs, hd, ea, w1s, w1d, w1e, b1[None, :],
                        W2.astype(bf), b2[None, :], Wsrc.astype(bf),
                        bsrc[None, :])
        aggs.append(_sc_scatter(off, ek, msg, dst_idx, zeros_nh))

    cnt = (hps[0][0, 0] + hps[0][1, 0] + hps[1][0, 0] + hps[1][1, 0]
           + hps[2][0, 0] + hps[2][1, 0])[:, None]
    out = _node_update(h2, aggs[0], aggs[1], aggs[2], cnt,
                       Wself, bself[None, :], Wagg, bagg[None, :],
                       gamma[None, :], beta[None, :])
    return out[None]
